# baseline (speedup 1.0000x reference)
import functools

import jax
import jax.numpy as jnp
from jax import lax
from jax.experimental import pallas as pl
from jax.experimental.pallas import tpu as pltpu
from jax.experimental.pallas import tpu_sc as plsc

N_NODES = 10000
N_EDGES = 320000
CH = 128
HCH = 64
NG = 64
KB = 80
NB = N_EDGES // KB
NSUB = 16
ROWS_PER_TILE = N_NODES // NSUB
GPW = NG // NSUB

_MESH = plsc.VectorSubcoreMesh(core_axis_name="c", subcore_axis_name="s")


NSLOT = 6
NBLK = NB // NSUB
NMACRO = (NBLK + 2 + NSLOT - 1) // NSLOT


def _sc_aggr_body(xs, gsrc, dsts, ew, tv, zrows, out,
                  gidx0, gidx1, ewb0, ewb1, rows0, rows1,
                  dstb0, dstb1, dstb2, stg0, stg1, stg2, t_v, acc,
                  si0, si1, sg0, sg1, sd0, sd1, sd2, ss0, ss1, ss2):
    gidx = [gidx0, gidx1]
    ewb = [ewb0, ewb1]
    rows = [rows0, rows1]
    dstb = [dstb0, dstb1, dstb2]
    stg = [stg0, stg1, stg2]
    si = [si0, si1]
    sg = [sg0, sg1]
    sd = [sd0, sd1, sd2]
    ss = [ss0, ss1, ss2]

    c = lax.axis_index("c")
    s = lax.axis_index("s")
    pltpu.sync_copy(zrows, acc.at[pl.ds(s * ROWS_PER_TILE, ROWS_PER_TILE)])
    pltpu.sync_copy(tv, t_v)
    plsc.subcore_barrier()

    ts = t_v[...][0]
    nblk = NBLK

    def base(i):
        return (s + i * NSUB) * KB

    def issue_idx(i, f):
        pltpu.async_copy(gsrc.at[c, pl.ds(base(i), KB)], gidx[f], si[f])
        pltpu.async_copy(ew.at[pl.ds(base(i), KB)], ewb[f], si[f])

    def wait_idx(i, f):
        pltpu.make_async_copy(gsrc.at[c, pl.ds(base(i), KB)], gidx[f], si[f]).wait()
        pltpu.make_async_copy(ew.at[pl.ds(base(i), KB)], ewb[f], si[f]).wait()

    issue_idx(0, 0)
    issue_idx(1, 1)
    wait_idx(0, 0)
    pltpu.async_copy(xs.at[gidx[0]], rows[0], sg[0])
    pltpu.async_copy(dsts.at[pl.ds(base(0), KB)], dstb[0], sd[0])

    def macro_body(ii, carry):
        for kk in range(NSLOT):
            i = ii * NSLOT + kk
            f = kk % 2
            f1 = (kk + 1) % 2
            b = kk % 3
            b1 = (kk + 1) % 3
            bm2 = (kk - 2) % 3

            @pl.when((i >= 2) & (i - 2 < nblk))
            def _():
                pltpu.make_async_copy(stg[bm2], acc.at[dstb[bm2]], ss[bm2]).wait()

            @pl.when(i < nblk)
            def _():
                pltpu.make_async_copy(xs.at[gidx[f]], rows[f], sg[f]).wait()

                @plsc.parallel_loop(0, KB // 16, unroll=2)
                def _grp(eb):
                    ew16 = ewb[f][pl.ds(eb * 16, 16)]
                    for k in range(16):
                        e = eb * 16 + k
                        w = ew16[k]
                        wt = w * ts
                        rvs = [rows[f][e, pl.ds(j * 16, 16)]
                               for j in range(HCH // 16)]
                        exs = [jnp.exp(r * wt) for r in rvs]
                        for j in range(HCH // 16):
                            stg[b][e, pl.ds(HCH + j * 16, 16)] = exs[j]
                            stg[b][e, pl.ds(j * 16, 16)] = (rvs[j] * w) * exs[j]

                pltpu.make_async_copy(dsts.at[pl.ds(base(i), KB)], dstb[b], sd[b]).wait()
                pltpu.async_copy(stg[b], acc.at[dstb[b]], ss[b], add=True)

            @pl.when(i + 1 < nblk)
            def _():
                pltpu.async_copy(dsts.at[pl.ds(base(i + 1), KB)], dstb[b1], sd[b1])

            @pl.when(i + 2 < nblk)
            def _():
                issue_idx(i + 2, f)

            @pl.when(i + 1 < nblk)
            def _():
                wait_idx(i + 1, f1)
                pltpu.async_copy(xs.at[gidx[f1]], rows[f1], sg[f1])
        return carry

    lax.fori_loop(0, NMACRO, macro_body, 0)
    plsc.subcore_barrier()
    pltpu.sync_copy(acc.at[pl.ds(s * ROWS_PER_TILE, ROWS_PER_TILE)],
                    out.at[pl.ds(c * N_NODES + s * ROWS_PER_TILE, ROWS_PER_TILE)])


_sc_aggr = functools.partial(
    pl.kernel,
    _sc_aggr_body,
    out_type=jax.ShapeDtypeStruct((2 * N_NODES, CH), jnp.float32),
    mesh=_MESH,
    scratch_types=[
        pltpu.VMEM((KB,), jnp.int32),
        pltpu.VMEM((KB,), jnp.int32),
        pltpu.VMEM((KB,), jnp.float32),
        pltpu.VMEM((KB,), jnp.float32),
        pltpu.VMEM((KB, HCH), jnp.float32),
        pltpu.VMEM((KB, HCH), jnp.float32),
        pltpu.VMEM((KB,), jnp.int32),
        pltpu.VMEM((KB,), jnp.int32),
        pltpu.VMEM((KB,), jnp.int32),
        pltpu.VMEM((KB, CH), jnp.float32),
        pltpu.VMEM((KB, CH), jnp.float32),
        pltpu.VMEM((KB, CH), jnp.float32),
        pltpu.VMEM((16,), jnp.float32),
        pltpu.VMEM_SHARED((N_NODES, CH), jnp.float32),
        pltpu.SemaphoreType.DMA,
        pltpu.SemaphoreType.DMA,
        pltpu.SemaphoreType.DMA,
        pltpu.SemaphoreType.DMA,
        pltpu.SemaphoreType.DMA,
        pltpu.SemaphoreType.DMA,
        pltpu.SemaphoreType.DMA,
        pltpu.SemaphoreType.DMA,
        pltpu.SemaphoreType.DMA,
        pltpu.SemaphoreType.DMA,
    ],
    compiler_params=pltpu.CompilerParams(use_tc_tiling_on_sc=False),
)()


def _sc_pool_body(xs1, xs2, xs3, offs, pooled, offs_v, buf8, buf1, out_v):
    c = lax.axis_index("c")
    s = lax.axis_index("s")
    pltpu.sync_copy(offs, offs_v)
    owin = offs_v[s, pl.ds(0, 16)]
    neg = jnp.full((16,), -jnp.inf, dtype=jnp.float32)

    for l, xsl in enumerate((xs1, xs2, xs3)):
        for k in range(GPW):
            g = s * GPW + k
            start = owin[k]
            end = owin[k + 1]
            row0 = c * N_NODES + start
            length = end - start
            nfull = length // 8

            def chunk_body(i, m):
                pltpu.sync_copy(xsl.at[pl.ds(row0 + i * 8, 8)], buf8)
                for r in range(8):
                    m = tuple(
                        jnp.maximum(m[j], buf8[r, pl.ds(j * 16, 16)])
                        for j in range(HCH // 16))
                return m

            def rem_body(i, m):
                pltpu.sync_copy(xsl.at[pl.ds(row0 + nfull * 8 + i, 1)], buf1)
                return tuple(
                    jnp.maximum(m[j], buf1[0, pl.ds(j * 16, 16)])
                    for j in range(HCH // 16))

            m = (neg, neg, neg, neg)
            m = lax.fori_loop(0, nfull, chunk_body, m)
            m = lax.fori_loop(0, length - nfull * 8, rem_body, m)
            for j in range(HCH // 16):
                out_v[0, pl.ds(j * 16, 16)] = jnp.where(length > 0, m[j], 0.0)
            idx = l * 2 * NG + c * NG + g
            pltpu.sync_copy(out_v, pooled.at[pl.ds(idx, 1)])


_sc_pool = functools.partial(
    pl.kernel,
    _sc_pool_body,
    out_type=jax.ShapeDtypeStruct((3 * 2 * NG, HCH), jnp.float32),
    mesh=_MESH,
    scratch_types=[
        pltpu.VMEM((NSUB, 16), jnp.int32),
        pltpu.VMEM((8, HCH), jnp.float32),
        pltpu.VMEM((1, HCH), jnp.float32),
        pltpu.VMEM((1, HCH), jnp.float32),
    ],
    compiler_params=pltpu.CompilerParams(use_tc_tiling_on_sc=False),
)()


def _tc_layer_body(nd_ref, xs_ref, wrel_ref, brel_ref, wroot_ref, out_ref):
    nd0 = nd_ref[0]
    nd1 = nd_ref[1]
    num = jnp.concatenate([nd0[:, :HCH], nd1[:, :HCH]], axis=1)
    den = jnp.concatenate([nd0[:, HCH:], nd1[:, HCH:]], axis=1)
    aggr = jnp.where(den > 0.0, num / jnp.where(den > 0.0, den, 1.0), 0.0)
    x = jnp.concatenate([xs_ref[0], xs_ref[1]], axis=1)
    h = (jnp.dot(aggr, wrel_ref[...], preferred_element_type=jnp.float32,
                 precision=lax.Precision.HIGHEST)
         + jnp.dot(x, wroot_ref[...], preferred_element_type=jnp.float32,
                   precision=lax.Precision.HIGHEST)
         + brel_ref[...])
    mu = jnp.mean(h, axis=1, keepdims=True)
    hc = h - mu
    var = jnp.mean(hc * hc, axis=1, keepdims=True)
    y = jnp.maximum(hc * lax.rsqrt(var + 1e-5), 0.0)
    out_ref[0] = y[:, :HCH]
    out_ref[1] = y[:, HCH:]


def _tc_layer(nd, xs, wrel, brel, wroot):
    R = 1000
    return pl.pallas_call(
        _tc_layer_body,
        grid=(N_NODES // R,),
        in_specs=[
            pl.BlockSpec((2, R, CH), lambda i: (0, i, 0)),
            pl.BlockSpec((2, R, HCH), lambda i: (0, i, 0)),
            pl.BlockSpec((CH, CH), lambda i: (0, 0)),
            pl.BlockSpec((1, CH), lambda i: (0, 0)),
            pl.BlockSpec((CH, CH), lambda i: (0, 0)),
        ],
        out_specs=pl.BlockSpec((2, R, HCH), lambda i: (0, i, 0)),
        out_shape=jax.ShapeDtypeStruct((2, N_NODES, HCH), jnp.float32),
    )(nd, xs, wrel, brel, wroot)


def _tc_mlp_body(p_ref, w1_ref, b1_ref, w2_ref, b2_ref, out_ref):
    p = p_ref[...]
    hcat = jnp.concatenate(
        [p[0, 0], p[0, 1], p[1, 0], p[1, 1], p[2, 0], p[2, 1]], axis=1)
    h = jnp.maximum(
        jnp.dot(hcat, w1_ref[...], preferred_element_type=jnp.float32,
                precision=lax.Precision.HIGHEST) + b1_ref[...], 0.0)
    o = (jnp.dot(h, w2_ref[...], preferred_element_type=jnp.float32,
                 precision=lax.Precision.HIGHEST) + b2_ref[...])
    nrm = jnp.maximum(jnp.sqrt(jnp.sum(o * o, axis=1, keepdims=True)), 1e-12)
    out_ref[...] = o / nrm


def _tc_mlp(p, w1, b1, w2, b2):
    return pl.pallas_call(
        _tc_mlp_body,
        out_shape=jax.ShapeDtypeStruct((NG, HCH), jnp.float32),
    )(p, w1, b1, w2, b2)


def kernel(x, edge_index, batch, edge_weight, W1_rel, b1_rel, W1_root,
           W2_rel, b2_rel, W2_root, W3_rel, b3_rel, W3_root, t,
           W_lin1, b_lin1, W_lin2, b_lin2):
    srcs = edge_index[0]
    dsts = edge_index[1]
    gsrc = jnp.stack([srcs, srcs + N_NODES])
    tv = jnp.full((16,), t, dtype=jnp.float32)
    zrows = jnp.zeros((ROWS_PER_TILE, CH), jnp.float32)
    offs = jnp.searchsorted(batch, jnp.arange(NG + 1, dtype=jnp.int32)).astype(jnp.int32)
    win_idx = jnp.minimum(jnp.arange(NSUB)[:, None] * GPW + jnp.arange(16)[None, :], NG)
    offs_wins = offs[win_idx]

    xs = jnp.reshape(jnp.transpose(jnp.reshape(x, (N_NODES, 2, HCH)), (1, 0, 2)),
                     (2 * N_NODES, HCH))
    layer_feats = []
    for (wrel, brel, wroot) in ((W1_rel, b1_rel, W1_root),
                                (W2_rel, b2_rel, W2_root),
                                (W3_rel, b3_rel, W3_root)):
        nd = _sc_aggr(xs, gsrc, dsts, edge_weight, tv, zrows)
        xs3d = _tc_layer(jnp.reshape(nd, (2, N_NODES, CH)),
                         jnp.reshape(xs, (2, N_NODES, HCH)),
                         wrel, jnp.reshape(brel, (1, CH)), wroot)
        xs = jnp.reshape(xs3d, (2 * N_NODES, HCH))
        layer_feats.append(xs)

    pooled = _sc_pool(layer_feats[0], layer_feats[1], layer_feats[2], offs_wins)
    out = _tc_mlp(jnp.reshape(pooled, (3, 2, NG, HCH)),
                  W_lin1, jnp.reshape(b_lin1, (1, 2 * CH)),
                  W_lin2, jnp.reshape(b_lin2, (1, HCH)))
    return out


# speedup vs baseline: 1.5113x; 1.5113x over previous
import functools

import jax
import jax.numpy as jnp
from jax import lax
from jax.experimental import pallas as pl
from jax.experimental.pallas import tpu as pltpu
from jax.experimental.pallas import tpu_sc as plsc

N_NODES = 10000
N_EDGES = 320000
CH = 128
HCH = 64
NG = 64
KB = 80
NB = N_EDGES // KB
NSUB = 16
ROWS_PER_TILE = N_NODES // NSUB
GPW = NG // NSUB

_MESH = plsc.VectorSubcoreMesh(core_axis_name="c", subcore_axis_name="s")


NSLOT = 6
NBLK = NB // NSUB
NMACRO = (NBLK + 2 + NSLOT - 1) // NSLOT


def _sc_aggr_body(xs, gsrc, dsts, ew, tv, zrows, out,
                  gidx0, gidx1, ewb0, ewb1, rows0, rows1,
                  dstb0, dstb1, dstb2, stg0, stg1, stg2, t_v, acc,
                  si0, si1, sg0, sg1, sd0, sd1, sd2, ss0, ss1, ss2):
    gidx = [gidx0, gidx1]
    ewb = [ewb0, ewb1]
    rows = [rows0, rows1]
    dstb = [dstb0, dstb1, dstb2]
    stg = [stg0, stg1, stg2]
    si = [si0, si1]
    sg = [sg0, sg1]
    sd = [sd0, sd1, sd2]
    ss = [ss0, ss1, ss2]

    c = lax.axis_index("c")
    s = lax.axis_index("s")
    pltpu.sync_copy(zrows, acc.at[pl.ds(s * ROWS_PER_TILE, ROWS_PER_TILE)])
    pltpu.sync_copy(tv, t_v)
    plsc.subcore_barrier()

    ts = t_v[...][0]
    nblk = NBLK

    def base(i):
        return (s + i * NSUB) * KB

    def issue_idx(i, f):
        pltpu.async_copy(gsrc.at[c, pl.ds(base(i), KB)], gidx[f], si[f])
        pltpu.async_copy(ew.at[pl.ds(base(i), KB)], ewb[f], si[f])

    def wait_idx(i, f):
        pltpu.make_async_copy(gsrc.at[c, pl.ds(base(i), KB)], gidx[f], si[f]).wait()
        pltpu.make_async_copy(ew.at[pl.ds(base(i), KB)], ewb[f], si[f]).wait()

    issue_idx(0, 0)
    issue_idx(1, 1)
    wait_idx(0, 0)
    pltpu.async_copy(xs.at[gidx[0]], rows[0], sg[0])
    pltpu.async_copy(dsts.at[pl.ds(base(0), KB)], dstb[0], sd[0])

    def macro_body(ii, carry):
        for kk in range(NSLOT):
            i = ii * NSLOT + kk
            f = kk % 2
            f1 = (kk + 1) % 2
            b = kk % 3
            b1 = (kk + 1) % 3
            bm2 = (kk - 2) % 3

            @pl.when((i >= 2) & (i - 2 < nblk))
            def _():
                pltpu.make_async_copy(stg[bm2], acc.at[dstb[bm2]], ss[bm2]).wait()

            @pl.when(i < nblk)
            def _():
                pltpu.make_async_copy(xs.at[gidx[f]], rows[f], sg[f]).wait()

                @plsc.parallel_loop(0, KB // 16)
                def _grp(eb):
                    ew16 = ewb[f][pl.ds(eb * 16, 16)]
                    for k in range(16):
                        e = eb * 16 + k
                        w = ew16[k]
                        wt = w * ts
                        rvs = [rows[f][e, pl.ds(j * 16, 16)]
                               for j in range(HCH // 16)]
                        exs = [jnp.exp(r * wt) for r in rvs]
                        for j in range(HCH // 16):
                            stg[b][e, pl.ds(HCH + j * 16, 16)] = exs[j]
                            stg[b][e, pl.ds(j * 16, 16)] = (rvs[j] * w) * exs[j]

                pltpu.make_async_copy(dsts.at[pl.ds(base(i), KB)], dstb[b], sd[b]).wait()
                pltpu.async_copy(stg[b], acc.at[dstb[b]], ss[b], add=True)

            @pl.when(i + 1 < nblk)
            def _():
                pltpu.async_copy(dsts.at[pl.ds(base(i + 1), KB)], dstb[b1], sd[b1])

            @pl.when(i + 2 < nblk)
            def _():
                issue_idx(i + 2, f)

            @pl.when(i + 1 < nblk)
            def _():
                wait_idx(i + 1, f1)
                pltpu.async_copy(xs.at[gidx[f1]], rows[f1], sg[f1])
        return carry

    lax.fori_loop(0, NMACRO, macro_body, 0)
    plsc.subcore_barrier()
    pltpu.sync_copy(acc.at[pl.ds(s * ROWS_PER_TILE, ROWS_PER_TILE)],
                    out.at[pl.ds(c * N_NODES + s * ROWS_PER_TILE, ROWS_PER_TILE)])


_sc_aggr = functools.partial(
    pl.kernel,
    _sc_aggr_body,
    out_type=jax.ShapeDtypeStruct((2 * N_NODES, CH), jnp.float32),
    mesh=_MESH,
    scratch_types=[
        pltpu.VMEM((KB,), jnp.int32),
        pltpu.VMEM((KB,), jnp.int32),
        pltpu.VMEM((KB,), jnp.float32),
        pltpu.VMEM((KB,), jnp.float32),
        pltpu.VMEM((KB, HCH), jnp.float32),
        pltpu.VMEM((KB, HCH), jnp.float32),
        pltpu.VMEM((KB,), jnp.int32),
        pltpu.VMEM((KB,), jnp.int32),
        pltpu.VMEM((KB,), jnp.int32),
        pltpu.VMEM((KB, CH), jnp.float32),
        pltpu.VMEM((KB, CH), jnp.float32),
        pltpu.VMEM((KB, CH), jnp.float32),
        pltpu.VMEM((16,), jnp.float32),
        pltpu.VMEM_SHARED((N_NODES, CH), jnp.float32),
        pltpu.SemaphoreType.DMA,
        pltpu.SemaphoreType.DMA,
        pltpu.SemaphoreType.DMA,
        pltpu.SemaphoreType.DMA,
        pltpu.SemaphoreType.DMA,
        pltpu.SemaphoreType.DMA,
        pltpu.SemaphoreType.DMA,
        pltpu.SemaphoreType.DMA,
        pltpu.SemaphoreType.DMA,
        pltpu.SemaphoreType.DMA,
    ],
    compiler_params=pltpu.CompilerParams(use_tc_tiling_on_sc=False),
)()


def _sc_pool_body(xs1, xs2, xs3, offs, pooled, offs_v, buf64, buf8, buf1, out_v,
                  sp0, sp1, sp2):
    c = lax.axis_index("c")
    s = lax.axis_index("s")
    pltpu.sync_copy(offs, offs_v)
    owin = offs_v[s, pl.ds(0, 16)]
    neg = jnp.full((16,), -jnp.inf, dtype=jnp.float32)
    xsl = (xs1, xs2, xs3)
    sps = (sp0, sp1, sp2)
    NV = HCH // 16

    for k in range(GPW):
        g = s * GPW + k
        start = owin[k]
        end = owin[k + 1]
        row0 = c * N_NODES + start
        length = end - start
        n64 = length // 64
        rem64 = length - n64 * 64
        n8 = rem64 // 8
        rem8 = rem64 - n8 * 8

        def chunk(m, rbase, cnt, buf):
            for l in range(3):
                pltpu.async_copy(xsl[l].at[pl.ds(rbase, cnt)], buf.at[l], sps[l])
            for l in range(3):
                pltpu.make_async_copy(xsl[l].at[pl.ds(rbase, cnt)], buf.at[l],
                                      sps[l]).wait()

            def row_body(r, mm):
                return tuple(
                    jnp.maximum(mm[l * NV + j], buf[l, r, pl.ds(j * 16, 16)])
                    for l in range(3) for j in range(NV))
            return lax.fori_loop(0, cnt, row_body, m)

        def c64(i, m):
            return chunk(m, row0 + i * 64, 64, buf64)

        def c8(i, m):
            return chunk(m, row0 + n64 * 64 + i * 8, 8, buf8)

        def c1(i, m):
            return chunk(m, row0 + n64 * 64 + n8 * 8 + i, 1, buf1)

        m = lax.fori_loop(0, n64, c64, (neg,) * (3 * NV))
        m = lax.fori_loop(0, n8, c8, m)
        m = lax.fori_loop(0, rem8, c1, m)

        for l in range(3):
            for j in range(NV):
                out_v[0, pl.ds(j * 16, 16)] = jnp.where(length > 0,
                                                        m[l * NV + j], 0.0)
            idx = l * 2 * NG + c * NG + g
            pltpu.sync_copy(out_v, pooled.at[pl.ds(idx, 1)])


_sc_pool = functools.partial(
    pl.kernel,
    _sc_pool_body,
    out_type=jax.ShapeDtypeStruct((3 * 2 * NG, HCH), jnp.float32),
    mesh=_MESH,
    scratch_types=[
        pltpu.VMEM((NSUB, 16), jnp.int32),
        pltpu.VMEM((3, 64, HCH), jnp.float32),
        pltpu.VMEM((3, 8, HCH), jnp.float32),
        pltpu.VMEM((3, 1, HCH), jnp.float32),
        pltpu.VMEM((1, HCH), jnp.float32),
        pltpu.SemaphoreType.DMA,
        pltpu.SemaphoreType.DMA,
        pltpu.SemaphoreType.DMA,
    ],
    compiler_params=pltpu.CompilerParams(use_tc_tiling_on_sc=False),
)()


def _tc_layer_body(nd_ref, xs_ref, wrel_ref, brel_ref, wroot_ref, out_ref):
    nd0 = nd_ref[0]
    nd1 = nd_ref[1]
    num = jnp.concatenate([nd0[:, :HCH], nd1[:, :HCH]], axis=1)
    den = jnp.concatenate([nd0[:, HCH:], nd1[:, HCH:]], axis=1)
    aggr = jnp.where(den > 0.0, num / jnp.where(den > 0.0, den, 1.0), 0.0)
    x = jnp.concatenate([xs_ref[0], xs_ref[1]], axis=1)
    h = (jnp.dot(aggr, wrel_ref[...], preferred_element_type=jnp.float32,
                 precision=lax.Precision.HIGHEST)
         + jnp.dot(x, wroot_ref[...], preferred_element_type=jnp.float32,
                   precision=lax.Precision.HIGHEST)
         + brel_ref[...])
    mu = jnp.mean(h, axis=1, keepdims=True)
    hc = h - mu
    var = jnp.mean(hc * hc, axis=1, keepdims=True)
    y = jnp.maximum(hc * lax.rsqrt(var + 1e-5), 0.0)
    out_ref[0] = y[:, :HCH]
    out_ref[1] = y[:, HCH:]


def _tc_layer(nd, xs, wrel, brel, wroot):
    R = 1000
    return pl.pallas_call(
        _tc_layer_body,
        grid=(N_NODES // R,),
        in_specs=[
            pl.BlockSpec((2, R, CH), lambda i: (0, i, 0)),
            pl.BlockSpec((2, R, HCH), lambda i: (0, i, 0)),
            pl.BlockSpec((CH, CH), lambda i: (0, 0)),
            pl.BlockSpec((1, CH), lambda i: (0, 0)),
            pl.BlockSpec((CH, CH), lambda i: (0, 0)),
        ],
        out_specs=pl.BlockSpec((2, R, HCH), lambda i: (0, i, 0)),
        out_shape=jax.ShapeDtypeStruct((2, N_NODES, HCH), jnp.float32),
    )(nd, xs, wrel, brel, wroot)


def _tc_mlp_body(p_ref, w1_ref, b1_ref, w2_ref, b2_ref, out_ref):
    p = p_ref[...]
    hcat = jnp.concatenate(
        [p[0, 0], p[0, 1], p[1, 0], p[1, 1], p[2, 0], p[2, 1]], axis=1)
    h = jnp.maximum(
        jnp.dot(hcat, w1_ref[...], preferred_element_type=jnp.float32,
                precision=lax.Precision.HIGHEST) + b1_ref[...], 0.0)
    o = (jnp.dot(h, w2_ref[...], preferred_element_type=jnp.float32,
                 precision=lax.Precision.HIGHEST) + b2_ref[...])
    nrm = jnp.maximum(jnp.sqrt(jnp.sum(o * o, axis=1, keepdims=True)), 1e-12)
    out_ref[...] = o / nrm


def _tc_mlp(p, w1, b1, w2, b2):
    return pl.pallas_call(
        _tc_mlp_body,
        out_shape=jax.ShapeDtypeStruct((NG, HCH), jnp.float32),
    )(p, w1, b1, w2, b2)


def kernel(x, edge_index, batch, edge_weight, W1_rel, b1_rel, W1_root,
           W2_rel, b2_rel, W2_root, W3_rel, b3_rel, W3_root, t,
           W_lin1, b_lin1, W_lin2, b_lin2):
    srcs = edge_index[0]
    dsts = edge_index[1]
    gsrc = jnp.stack([srcs, srcs + N_NODES])
    tv = jnp.full((16,), t, dtype=jnp.float32)
    zrows = jnp.zeros((ROWS_PER_TILE, CH), jnp.float32)
    offs = jnp.searchsorted(batch, jnp.arange(NG + 1, dtype=jnp.int32)).astype(jnp.int32)
    win_idx = jnp.minimum(jnp.arange(NSUB)[:, None] * GPW + jnp.arange(16)[None, :], NG)
    offs_wins = offs[win_idx]

    xs = jnp.reshape(jnp.transpose(jnp.reshape(x, (N_NODES, 2, HCH)), (1, 0, 2)),
                     (2 * N_NODES, HCH))
    layer_feats = []
    for (wrel, brel, wroot) in ((W1_rel, b1_rel, W1_root),
                                (W2_rel, b2_rel, W2_root),
                                (W3_rel, b3_rel, W3_root)):
        nd = _sc_aggr(xs, gsrc, dsts, edge_weight, tv, zrows)
        xs3d = _tc_layer(jnp.reshape(nd, (2, N_NODES, CH)),
                         jnp.reshape(xs, (2, N_NODES, HCH)),
                         wrel, jnp.reshape(brel, (1, CH)), wroot)
        xs = jnp.reshape(xs3d, (2 * N_NODES, HCH))
        layer_feats.append(xs)

    pooled = _sc_pool(layer_feats[0], layer_feats[1], layer_feats[2], offs_wins)
    out = _tc_mlp(jnp.reshape(pooled, (3, 2, NG, HCH)),
                  W_lin1, jnp.reshape(b_lin1, (1, 2 * CH)),
                  W_lin2, jnp.reshape(b_lin2, (1, HCH)))
    return out


# speedup vs baseline: 1.7050x; 1.1282x over previous
import functools

import jax
import jax.numpy as jnp
from jax import lax
from jax.experimental import pallas as pl
from jax.experimental.pallas import tpu as pltpu
from jax.experimental.pallas import tpu_sc as plsc

N_NODES = 10000
N_EDGES = 320000
CH = 128
HCH = 64
NG = 64
KB = 80
NB = N_EDGES // KB
NSUB = 16
ROWS_PER_TILE = N_NODES // NSUB
GPW = NG // NSUB

_MESH = plsc.VectorSubcoreMesh(core_axis_name="c", subcore_axis_name="s")


NSLOT = 6
NBLK = NB // NSUB
NMACRO = (NBLK + 2 + NSLOT - 1) // NSLOT


def _sc_aggr_body(xs, gsrc, dsts, ew, tv, zrows, out,
                  gidx0, gidx1, ewb0, ewb1, rows0, rows1,
                  dstb0, dstb1, dstb2, stg0, stg1, stg2, t_v, acc,
                  si0, si1, sg0, sg1, sd0, sd1, sd2, ss0, ss1, ss2):
    gidx = [gidx0, gidx1]
    ewb = [ewb0, ewb1]
    rows = [rows0, rows1]
    dstb = [dstb0, dstb1, dstb2]
    stg = [stg0, stg1, stg2]
    si = [si0, si1]
    sg = [sg0, sg1]
    sd = [sd0, sd1, sd2]
    ss = [ss0, ss1, ss2]

    c = lax.axis_index("c")
    s = lax.axis_index("s")
    pltpu.sync_copy(zrows, acc.at[pl.ds(s * ROWS_PER_TILE, ROWS_PER_TILE)])
    pltpu.sync_copy(tv, t_v)
    plsc.subcore_barrier()

    ts = t_v[...][0]
    nblk = NBLK

    def base(i):
        return (s + i * NSUB) * KB

    def issue_idx(i, f):
        pltpu.async_copy(gsrc.at[c, pl.ds(base(i), KB)], gidx[f], si[f])
        pltpu.async_copy(ew.at[pl.ds(base(i), KB)], ewb[f], si[f])

    def wait_idx(i, f):
        pltpu.make_async_copy(gsrc.at[c, pl.ds(base(i), KB)], gidx[f], si[f]).wait()
        pltpu.make_async_copy(ew.at[pl.ds(base(i), KB)], ewb[f], si[f]).wait()

    issue_idx(0, 0)
    issue_idx(1, 1)
    wait_idx(0, 0)
    pltpu.async_copy(xs.at[gidx[0]], rows[0], sg[0])
    pltpu.async_copy(dsts.at[pl.ds(base(0), KB)], dstb[0], sd[0])

    def macro_body(ii, carry):
        for kk in range(NSLOT):
            i = ii * NSLOT + kk
            f = kk % 2
            f1 = (kk + 1) % 2
            b = kk % 3
            b1 = (kk + 1) % 3
            bm2 = (kk - 2) % 3

            @pl.when((i >= 2) & (i - 2 < nblk))
            def _():
                pltpu.make_async_copy(stg[bm2], acc.at[dstb[bm2]], ss[bm2]).wait()

            @pl.when(i < nblk)
            def _():
                pltpu.make_async_copy(xs.at[gidx[f]], rows[f], sg[f]).wait()

                @plsc.parallel_loop(0, KB // 16)
                def _grp(eb):
                    ew16 = ewb[f][pl.ds(eb * 16, 16)]
                    NV = HCH // 16
                    for k2 in range(8):
                        es = (eb * 16 + 2 * k2, eb * 16 + 2 * k2 + 1)
                        ws = (ew16[2 * k2], ew16[2 * k2 + 1])
                        wts = (ws[0] * ts, ws[1] * ts)
                        rv = [[rows[f][es[p], pl.ds(j * 16, 16)]
                               for j in range(NV)] for p in range(2)]
                        exs = [[jnp.exp(rv[p][j] * wts[p]) for j in range(NV)]
                               for p in range(2)]
                        for j in range(NV):
                            for p in range(2):
                                stg[b][es[p], pl.ds(HCH + j * 16, 16)] = exs[p][j]
                                stg[b][es[p], pl.ds(j * 16, 16)] = \
                                    (rv[p][j] * ws[p]) * exs[p][j]

                pltpu.make_async_copy(dsts.at[pl.ds(base(i), KB)], dstb[b], sd[b]).wait()
                pltpu.async_copy(stg[b], acc.at[dstb[b]], ss[b], add=True)

            @pl.when(i + 1 < nblk)
            def _():
                pltpu.async_copy(dsts.at[pl.ds(base(i + 1), KB)], dstb[b1], sd[b1])

            @pl.when(i + 2 < nblk)
            def _():
                issue_idx(i + 2, f)

            @pl.when(i + 1 < nblk)
            def _():
                wait_idx(i + 1, f1)
                pltpu.async_copy(xs.at[gidx[f1]], rows[f1], sg[f1])
        return carry

    lax.fori_loop(0, NMACRO, macro_body, 0)
    plsc.subcore_barrier()
    pltpu.sync_copy(acc.at[pl.ds(s * ROWS_PER_TILE, ROWS_PER_TILE)],
                    out.at[pl.ds(c * N_NODES + s * ROWS_PER_TILE, ROWS_PER_TILE)])


_sc_aggr = functools.partial(
    pl.kernel,
    _sc_aggr_body,
    out_type=jax.ShapeDtypeStruct((2 * N_NODES, CH), jnp.float32),
    mesh=_MESH,
    scratch_types=[
        pltpu.VMEM((KB,), jnp.int32),
        pltpu.VMEM((KB,), jnp.int32),
        pltpu.VMEM((KB,), jnp.float32),
        pltpu.VMEM((KB,), jnp.float32),
        pltpu.VMEM((KB, HCH), jnp.float32),
        pltpu.VMEM((KB, HCH), jnp.float32),
        pltpu.VMEM((KB,), jnp.int32),
        pltpu.VMEM((KB,), jnp.int32),
        pltpu.VMEM((KB,), jnp.int32),
        pltpu.VMEM((KB, CH), jnp.float32),
        pltpu.VMEM((KB, CH), jnp.float32),
        pltpu.VMEM((KB, CH), jnp.float32),
        pltpu.VMEM((16,), jnp.float32),
        pltpu.VMEM_SHARED((N_NODES, CH), jnp.float32),
        pltpu.SemaphoreType.DMA,
        pltpu.SemaphoreType.DMA,
        pltpu.SemaphoreType.DMA,
        pltpu.SemaphoreType.DMA,
        pltpu.SemaphoreType.DMA,
        pltpu.SemaphoreType.DMA,
        pltpu.SemaphoreType.DMA,
        pltpu.SemaphoreType.DMA,
        pltpu.SemaphoreType.DMA,
        pltpu.SemaphoreType.DMA,
    ],
    compiler_params=pltpu.CompilerParams(use_tc_tiling_on_sc=False),
)()


def _sc_pool_body(xs1, xs2, xs3, offs, pooled, offs_v, buf64, buf8, buf1, out_v,
                  sp0, sp1, sp2):
    c = lax.axis_index("c")
    s = lax.axis_index("s")
    pltpu.sync_copy(offs, offs_v)
    owin = offs_v[s, pl.ds(0, 16)]
    neg = jnp.full((16,), -jnp.inf, dtype=jnp.float32)
    xsl = (xs1, xs2, xs3)
    sps = (sp0, sp1, sp2)
    NV = HCH // 16

    for k in range(GPW):
        g = s * GPW + k
        start = owin[k]
        end = owin[k + 1]
        row0 = c * N_NODES + start
        length = end - start
        n64 = length // 64
        rem64 = length - n64 * 64
        n8 = rem64 // 8
        rem8 = rem64 - n8 * 8

        def chunk(m, rbase, cnt, buf):
            for l in range(3):
                pltpu.async_copy(xsl[l].at[pl.ds(rbase, cnt)], buf.at[l], sps[l])
            for l in range(3):
                pltpu.make_async_copy(xsl[l].at[pl.ds(rbase, cnt)], buf.at[l],
                                      sps[l]).wait()

            def row_body(r, mm):
                return tuple(
                    jnp.maximum(mm[l * NV + j], buf[l, r, pl.ds(j * 16, 16)])
                    for l in range(3) for j in range(NV))
            return lax.fori_loop(0, cnt, row_body, m)

        def c64(i, m):
            return chunk(m, row0 + i * 64, 64, buf64)

        def c8(i, m):
            return chunk(m, row0 + n64 * 64 + i * 8, 8, buf8)

        def c1(i, m):
            return chunk(m, row0 + n64 * 64 + n8 * 8 + i, 1, buf1)

        m = lax.fori_loop(0, n64, c64, (neg,) * (3 * NV))
        m = lax.fori_loop(0, n8, c8, m)
        m = lax.fori_loop(0, rem8, c1, m)

        for l in range(3):
            for j in range(NV):
                out_v[0, pl.ds(j * 16, 16)] = jnp.where(length > 0,
                                                        m[l * NV + j], 0.0)
            idx = l * 2 * NG + c * NG + g
            pltpu.sync_copy(out_v, pooled.at[pl.ds(idx, 1)])


_sc_pool = functools.partial(
    pl.kernel,
    _sc_pool_body,
    out_type=jax.ShapeDtypeStruct((3 * 2 * NG, HCH), jnp.float32),
    mesh=_MESH,
    scratch_types=[
        pltpu.VMEM((NSUB, 16), jnp.int32),
        pltpu.VMEM((3, 64, HCH), jnp.float32),
        pltpu.VMEM((3, 8, HCH), jnp.float32),
        pltpu.VMEM((3, 1, HCH), jnp.float32),
        pltpu.VMEM((1, HCH), jnp.float32),
        pltpu.SemaphoreType.DMA,
        pltpu.SemaphoreType.DMA,
        pltpu.SemaphoreType.DMA,
    ],
    compiler_params=pltpu.CompilerParams(use_tc_tiling_on_sc=False),
)()


def _tc_layer_body(nd_ref, xs_ref, wrel_ref, brel_ref, wroot_ref, out_ref):
    nd0 = nd_ref[0]
    nd1 = nd_ref[1]
    num = jnp.concatenate([nd0[:, :HCH], nd1[:, :HCH]], axis=1)
    den = jnp.concatenate([nd0[:, HCH:], nd1[:, HCH:]], axis=1)
    aggr = jnp.where(den > 0.0, num / jnp.where(den > 0.0, den, 1.0), 0.0)
    x = jnp.concatenate([xs_ref[0], xs_ref[1]], axis=1)
    h = (jnp.dot(aggr, wrel_ref[...], preferred_element_type=jnp.float32,
                 precision=lax.Precision.HIGHEST)
         + jnp.dot(x, wroot_ref[...], preferred_element_type=jnp.float32,
                   precision=lax.Precision.HIGHEST)
         + brel_ref[...])
    mu = jnp.mean(h, axis=1, keepdims=True)
    hc = h - mu
    var = jnp.mean(hc * hc, axis=1, keepdims=True)
    y = jnp.maximum(hc * lax.rsqrt(var + 1e-5), 0.0)
    out_ref[0] = y[:, :HCH]
    out_ref[1] = y[:, HCH:]


def _tc_layer(nd, xs, wrel, brel, wroot):
    R = 1000
    return pl.pallas_call(
        _tc_layer_body,
        grid=(N_NODES // R,),
        in_specs=[
            pl.BlockSpec((2, R, CH), lambda i: (0, i, 0)),
            pl.BlockSpec((2, R, HCH), lambda i: (0, i, 0)),
            pl.BlockSpec((CH, CH), lambda i: (0, 0)),
            pl.BlockSpec((1, CH), lambda i: (0, 0)),
            pl.BlockSpec((CH, CH), lambda i: (0, 0)),
        ],
        out_specs=pl.BlockSpec((2, R, HCH), lambda i: (0, i, 0)),
        out_shape=jax.ShapeDtypeStruct((2, N_NODES, HCH), jnp.float32),
    )(nd, xs, wrel, brel, wroot)


def _tc_mlp_body(p_ref, w1_ref, b1_ref, w2_ref, b2_ref, out_ref):
    p = p_ref[...]
    hcat = jnp.concatenate(
        [p[0, 0], p[0, 1], p[1, 0], p[1, 1], p[2, 0], p[2, 1]], axis=1)
    h = jnp.maximum(
        jnp.dot(hcat, w1_ref[...], preferred_element_type=jnp.float32,
                precision=lax.Precision.HIGHEST) + b1_ref[...], 0.0)
    o = (jnp.dot(h, w2_ref[...], preferred_element_type=jnp.float32,
                 precision=lax.Precision.HIGHEST) + b2_ref[...])
    nrm = jnp.maximum(jnp.sqrt(jnp.sum(o * o, axis=1, keepdims=True)), 1e-12)
    out_ref[...] = o / nrm


def _tc_mlp(p, w1, b1, w2, b2):
    return pl.pallas_call(
        _tc_mlp_body,
        out_shape=jax.ShapeDtypeStruct((NG, HCH), jnp.float32),
    )(p, w1, b1, w2, b2)


def kernel(x, edge_index, batch, edge_weight, W1_rel, b1_rel, W1_root,
           W2_rel, b2_rel, W2_root, W3_rel, b3_rel, W3_root, t,
           W_lin1, b_lin1, W_lin2, b_lin2):
    srcs = edge_index[0]
    dsts = edge_index[1]
    gsrc = jnp.stack([srcs, srcs + N_NODES])
    tv = jnp.full((16,), t, dtype=jnp.float32)
    zrows = jnp.zeros((ROWS_PER_TILE, CH), jnp.float32)
    offs = jnp.searchsorted(batch, jnp.arange(NG + 1, dtype=jnp.int32)).astype(jnp.int32)
    win_idx = jnp.minimum(jnp.arange(NSUB)[:, None] * GPW + jnp.arange(16)[None, :], NG)
    offs_wins = offs[win_idx]

    xs = jnp.reshape(jnp.transpose(jnp.reshape(x, (N_NODES, 2, HCH)), (1, 0, 2)),
                     (2 * N_NODES, HCH))
    layer_feats = []
    for (wrel, brel, wroot) in ((W1_rel, b1_rel, W1_root),
                                (W2_rel, b2_rel, W2_root),
                                (W3_rel, b3_rel, W3_root)):
        nd = _sc_aggr(xs, gsrc, dsts, edge_weight, tv, zrows)
        xs3d = _tc_layer(jnp.reshape(nd, (2, N_NODES, CH)),
                         jnp.reshape(xs, (2, N_NODES, HCH)),
                         wrel, jnp.reshape(brel, (1, CH)), wroot)
        xs = jnp.reshape(xs3d, (2 * N_NODES, HCH))
        layer_feats.append(xs)

    pooled = _sc_pool(layer_feats[0], layer_feats[1], layer_feats[2], offs_wins)
    out = _tc_mlp(jnp.reshape(pooled, (3, 2, NG, HCH)),
                  W_lin1, jnp.reshape(b_lin1, (1, 2 * CH)),
                  W_lin2, jnp.reshape(b_lin2, (1, HCH)))
    return out
